# baseline (speedup 1.0000x reference)
import functools
import jax
import jax.numpy as jnp
from jax import lax
from jax.experimental import pallas as pl
from jax.experimental.pallas import tpu as pltpu
from jax.experimental.pallas import tpu_sc as plsc

LENGTH = 512
FEATURES = 768
LANES = 16
NUM_CORES = 2
NUM_SUBCORES = 16
NUM_WORKERS = NUM_CORES * NUM_SUBCORES
ROWS_PER_W = LENGTH // NUM_WORKERS
CHUNKS = FEATURES // LANES
EPS = 1e-12


def _rsqrt(x):
    i = lax.bitcast_convert_type(x, jnp.int32)
    i = jnp.int32(0x5F3759DF) - lax.shift_right_arithmetic(i, 1)
    y = lax.bitcast_convert_type(i, jnp.float32)
    for _ in range(4):
        y = y * (jnp.float32(1.5) - jnp.float32(0.5) * x * y * y)
    return y


def _body(tokens_hbm, positions_hbm, types_hbm, wte_hbm, wpe_hbm, tte_hbm,
          lnw_hbm, lnb_hbm, out_hbm,
          tok_idx, pos_idx, typ_idx, tok_rows, pos_rows, typ_rows,
          lnw_v, lnb_v, sem):
    wid = lax.axis_index("s") * NUM_CORES + lax.axis_index("c")
    base = wid * ROWS_PER_W

    pltpu.sync_copy(tokens_hbm.at[pl.ds(base, ROWS_PER_W)], tok_idx)
    pltpu.sync_copy(positions_hbm.at[pl.ds(base, ROWS_PER_W)], pos_idx)
    pltpu.sync_copy(types_hbm.at[pl.ds(base, ROWS_PER_W)], typ_idx)

    c1 = pltpu.async_copy(wte_hbm.at[tok_idx], tok_rows, sem)
    c2 = pltpu.async_copy(wpe_hbm.at[pos_idx], pos_rows, sem)
    c3 = pltpu.async_copy(tte_hbm.at[typ_idx], typ_rows, sem)
    c4 = pltpu.async_copy(lnw_hbm, lnw_v, sem)
    c5 = pltpu.async_copy(lnb_hbm, lnb_v, sem)
    c1.wait(); c2.wait(); c3.wait(); c4.wait(); c5.wait()

    inv_n = jnp.float32(1.0 / FEATURES)
    zero = jnp.zeros((LANES,), jnp.float32)
    for r in range(ROWS_PER_W):
        def stat_body(c, carry, r=r):
            s, q = carry
            sl = pl.ds(c * LANES, LANES)
            x = tok_rows[r, sl] + pos_rows[r, sl] + typ_rows[r, sl]
            tok_rows[r, sl] = x
            return s + x, q + x * x

        s, q = lax.fori_loop(0, CHUNKS, stat_body, (zero, zero))
        total = jnp.sum(s, axis=0)
        totq = jnp.sum(q, axis=0)
        mean = total * inv_n
        var = totq * inv_n - mean * mean
        rstd = _rsqrt(var + jnp.float32(EPS))
        mean_v = jnp.full((LANES,), mean, jnp.float32)
        rstd_v = jnp.full((LANES,), rstd, jnp.float32)

        def norm_body(c, carry, r=r):
            sl = pl.ds(c * LANES, LANES)
            x = tok_rows[r, sl]
            tok_rows[r, sl] = (x - mean_v) * rstd_v * lnw_v[sl] + lnb_v[sl]
            return carry

        lax.fori_loop(0, CHUNKS, norm_body, 0)

    pltpu.sync_copy(tok_rows, out_hbm.at[pl.ds(base, ROWS_PER_W)])


@functools.partial(jax.jit, donate_argnums=())
def _run(tokens, positions, types, wte, wpe, tte, ln_w, ln_b):
    mesh = plsc.VectorSubcoreMesh(core_axis_name="c", subcore_axis_name="s")
    f = functools.partial(
        pl.kernel,
        out_type=jax.ShapeDtypeStruct((LENGTH, FEATURES), jnp.float32),
        mesh=mesh,
        scratch_types=[
            pltpu.VMEM((ROWS_PER_W,), jnp.int32),
            pltpu.VMEM((ROWS_PER_W,), jnp.int32),
            pltpu.VMEM((ROWS_PER_W,), jnp.int32),
            pltpu.VMEM((ROWS_PER_W, FEATURES), jnp.float32),
            pltpu.VMEM((ROWS_PER_W, FEATURES), jnp.float32),
            pltpu.VMEM((ROWS_PER_W, FEATURES), jnp.float32),
            pltpu.VMEM((FEATURES,), jnp.float32),
            pltpu.VMEM((FEATURES,), jnp.float32),
            pltpu.SemaphoreType.DMA,
        ],
        compiler_params=pltpu.CompilerParams(needs_layout_passes=False),
    )(_body)
    return f(tokens, positions, types, wte, wpe, tte, ln_w, ln_b)


def kernel(tokens, positions, types, wte, wpe, tte, ln_w, ln_b):
    return _run(tokens.astype(jnp.int32), positions.astype(jnp.int32),
                types.astype(jnp.int32), wte, wpe, tte, ln_w, ln_b)


# speedup vs baseline: 1.1597x; 1.1597x over previous
import functools
import jax
import jax.numpy as jnp
from jax import lax
from jax.experimental import pallas as pl
from jax.experimental.pallas import tpu as pltpu
from jax.experimental.pallas import tpu_sc as plsc

LENGTH = 512
FEATURES = 768
LANES = 16
NUM_CORES = 2
NUM_SUBCORES = 16
NUM_WORKERS = NUM_CORES * NUM_SUBCORES
ROWS_PER_W = LENGTH // NUM_WORKERS
CHUNKS = FEATURES // LANES
EPS = 1e-12


def _rsqrt(x):
    i = lax.bitcast_convert_type(x, jnp.int32)
    i = jnp.int32(0x5F3759DF) - lax.shift_right_arithmetic(i, 1)
    y = lax.bitcast_convert_type(i, jnp.float32)
    for _ in range(4):
        y = y * (jnp.float32(1.5) - jnp.float32(0.5) * x * y * y)
    return y


def _body(tokens_hbm, positions_hbm, types_hbm, wte_hbm, wpe_hbm, tte_hbm,
          lnw_hbm, lnb_hbm, out_hbm,
          tok_idx, pos_idx, typ_idx, tok_rows, pos_rows, typ_rows,
          lnw_v, lnb_v, sem):
    wid = lax.axis_index("s") * NUM_CORES + lax.axis_index("c")
    base = wid * ROWS_PER_W

    pltpu.sync_copy(tokens_hbm.at[pl.ds(base, ROWS_PER_W)], tok_idx)
    pltpu.sync_copy(positions_hbm.at[pl.ds(base, ROWS_PER_W)], pos_idx)
    pltpu.sync_copy(types_hbm.at[pl.ds(base, ROWS_PER_W)], typ_idx)

    c1 = pltpu.async_copy(wte_hbm.at[tok_idx], tok_rows, sem)
    c2 = pltpu.async_copy(wpe_hbm.at[pos_idx], pos_rows, sem)
    c3 = pltpu.async_copy(tte_hbm.at[typ_idx], typ_rows, sem)
    c4 = pltpu.async_copy(lnw_hbm, lnw_v, sem)
    c5 = pltpu.async_copy(lnb_hbm, lnb_v, sem)
    c1.wait(); c2.wait(); c3.wait(); c4.wait(); c5.wait()

    inv_n = jnp.float32(1.0 / FEATURES)
    zero = jnp.zeros((LANES,), jnp.float32)

    def row_fn(r, _):
        s = zero
        q = zero
        for c in range(CHUNKS):
            sl = pl.ds(c * LANES, LANES)
            x = tok_rows[r, sl] + pos_rows[r, sl] + typ_rows[r, sl]
            tok_rows[r, sl] = x
            s = s + x
            q = q + x * x
        mean = jnp.sum(s, axis=0) * inv_n
        var = jnp.sum(q, axis=0) * inv_n - mean * mean
        rstd = _rsqrt(var + jnp.float32(EPS))
        mean_v = jnp.full((LANES,), mean, jnp.float32)
        rstd_v = jnp.full((LANES,), rstd, jnp.float32)
        for c in range(CHUNKS):
            sl = pl.ds(c * LANES, LANES)
            x = tok_rows[r, sl]
            tok_rows[r, sl] = (x - mean_v) * rstd_v * lnw_v[sl] + lnb_v[sl]
        return 0

    lax.fori_loop(0, ROWS_PER_W, row_fn, 0)

    pltpu.sync_copy(tok_rows, out_hbm.at[pl.ds(base, ROWS_PER_W)])


@functools.partial(jax.jit, donate_argnums=())
def _run(tokens, positions, types, wte, wpe, tte, ln_w, ln_b):
    mesh = plsc.VectorSubcoreMesh(core_axis_name="c", subcore_axis_name="s")
    f = functools.partial(
        pl.kernel,
        out_type=jax.ShapeDtypeStruct((LENGTH, FEATURES), jnp.float32),
        mesh=mesh,
        scratch_types=[
            pltpu.VMEM((ROWS_PER_W,), jnp.int32),
            pltpu.VMEM((ROWS_PER_W,), jnp.int32),
            pltpu.VMEM((ROWS_PER_W,), jnp.int32),
            pltpu.VMEM((ROWS_PER_W, FEATURES), jnp.float32),
            pltpu.VMEM((ROWS_PER_W, FEATURES), jnp.float32),
            pltpu.VMEM((ROWS_PER_W, FEATURES), jnp.float32),
            pltpu.VMEM((FEATURES,), jnp.float32),
            pltpu.VMEM((FEATURES,), jnp.float32),
            pltpu.SemaphoreType.DMA,
        ],
        compiler_params=pltpu.CompilerParams(needs_layout_passes=False),
    )(_body)
    return f(tokens, positions, types, wte, wpe, tte, ln_w, ln_b)


def kernel(tokens, positions, types, wte, wpe, tte, ln_w, ln_b):
    return _run(tokens.astype(jnp.int32), positions.astype(jnp.int32),
                types.astype(jnp.int32), wte, wpe, tte, ln_w, ln_b)


# speedup vs baseline: 1.2162x; 1.0487x over previous
import functools
import jax
import jax.numpy as jnp
from jax import lax
from jax.experimental import pallas as pl
from jax.experimental.pallas import tpu as pltpu
from jax.experimental.pallas import tpu_sc as plsc

LENGTH = 512
FEATURES = 768
LANES = 16
NUM_CORES = 2
NUM_SUBCORES = 16
NUM_WORKERS = NUM_CORES * NUM_SUBCORES
ROWS_PER_W = LENGTH // NUM_WORKERS
HALF = ROWS_PER_W // 2
CHUNKS = FEATURES // LANES
EPS = 1e-12


def _rsqrt(x):
    i = lax.bitcast_convert_type(x, jnp.int32)
    i = jnp.int32(0x5F3759DF) - lax.shift_right_arithmetic(i, 1)
    y = lax.bitcast_convert_type(i, jnp.float32)
    for _ in range(3):
        y = y * (jnp.float32(1.5) - jnp.float32(0.5) * x * y * y)
    return y


def _body(tokens_hbm, positions_hbm, types_hbm, wte_hbm, wpe_hbm, tte_hbm,
          lnw_hbm, lnb_hbm, out_hbm,
          tok_idx, pos_idx, typ_idx, tok_rows, pos_rows, typ_rows,
          emb_rows, out_rows, lnw_v, lnb_v, sem, osem):
    wid = lax.axis_index("s") * NUM_CORES + lax.axis_index("c")
    base = wid * ROWS_PER_W

    pltpu.sync_copy(tokens_hbm.at[pl.ds(base, ROWS_PER_W)], tok_idx)
    pltpu.sync_copy(positions_hbm.at[pl.ds(base, ROWS_PER_W)], pos_idx)
    pltpu.sync_copy(types_hbm.at[pl.ds(base, ROWS_PER_W)], typ_idx)

    g = []
    for h in range(2):
        rs = pl.ds(h * HALF, HALF)
        g.append(pltpu.async_copy(wte_hbm.at[tok_idx.at[rs]], tok_rows.at[rs], sem))
        g.append(pltpu.async_copy(wpe_hbm.at[pos_idx.at[rs]], pos_rows.at[rs], sem))
        g.append(pltpu.async_copy(tte_hbm.at[typ_idx.at[rs]], typ_rows.at[rs], sem))
    g.append(pltpu.async_copy(lnw_hbm, lnw_v, sem))
    g.append(pltpu.async_copy(lnb_hbm, lnb_v, sem))
    for c in (g[0], g[1], g[2], g[6], g[7]):
        c.wait()

    inv_n = jnp.float32(1.0 / FEATURES)
    zero = jnp.zeros((LANES,), jnp.float32)

    def row_fn(r, _):
        s = [zero] * 4
        q = [zero] * 4
        for c in range(CHUNKS):
            sl = pl.ds(c * LANES, LANES)
            x = tok_rows[r, sl] + pos_rows[r, sl] + typ_rows[r, sl]
            emb_rows[r, sl] = x
            k = c % 4
            s[k] = s[k] + x
            q[k] = q[k] + x * x
        sv = (s[0] + s[1]) + (s[2] + s[3])
        qv = (q[0] + q[1]) + (q[2] + q[3])
        mean = jnp.sum(sv, axis=0) * inv_n
        var = jnp.sum(qv, axis=0) * inv_n - mean * mean
        rstd = _rsqrt(var + jnp.float32(EPS))
        mean_v = jnp.full((LANES,), mean, jnp.float32)
        rstd_v = jnp.full((LANES,), rstd, jnp.float32)
        for c in range(CHUNKS):
            sl = pl.ds(c * LANES, LANES)
            x = emb_rows[r, sl]
            out_rows[r, sl] = (x - mean_v) * rstd_v * lnw_v[sl] + lnb_v[sl]
        return 0

    def loop_fn(r, _):
        @pl.when(r == HALF)
        def _mid():
            pltpu.async_copy(out_rows.at[pl.ds(0, HALF)],
                             out_hbm.at[pl.ds(base, HALF)], osem)
            for c in (g[3], g[4], g[5]):
                c.wait()
        return row_fn(r, _)

    lax.fori_loop(0, ROWS_PER_W, loop_fn, 0)
    o2 = pltpu.async_copy(out_rows.at[pl.ds(HALF, HALF)],
                          out_hbm.at[pl.ds(base + HALF, HALF)], osem)
    pltpu.make_async_copy(out_rows.at[pl.ds(0, HALF)],
                          out_hbm.at[pl.ds(base, HALF)], osem).wait()
    o2.wait()


@functools.partial(jax.jit, donate_argnums=())
def _run(tokens, positions, types, wte, wpe, tte, ln_w, ln_b):
    mesh = plsc.VectorSubcoreMesh(core_axis_name="c", subcore_axis_name="s")
    f = functools.partial(
        pl.kernel,
        out_type=jax.ShapeDtypeStruct((LENGTH, FEATURES), jnp.float32),
        mesh=mesh,
        scratch_types=[
            pltpu.VMEM((ROWS_PER_W,), jnp.int32),
            pltpu.VMEM((ROWS_PER_W,), jnp.int32),
            pltpu.VMEM((ROWS_PER_W,), jnp.int32),
            pltpu.VMEM((ROWS_PER_W, FEATURES), jnp.float32),
            pltpu.VMEM((ROWS_PER_W, FEATURES), jnp.float32),
            pltpu.VMEM((ROWS_PER_W, FEATURES), jnp.float32),
            pltpu.VMEM((ROWS_PER_W, FEATURES), jnp.float32),
            pltpu.VMEM((ROWS_PER_W, FEATURES), jnp.float32),
            pltpu.VMEM((FEATURES,), jnp.float32),
            pltpu.VMEM((FEATURES,), jnp.float32),
            pltpu.SemaphoreType.DMA,
            pltpu.SemaphoreType.DMA,
        ],
        compiler_params=pltpu.CompilerParams(needs_layout_passes=False),
    )(_body)
    return f(tokens, positions, types, wte, wpe, tte, ln_w, ln_b)


def kernel(tokens, positions, types, wte, wpe, tte, ln_w, ln_b):
    return _run(tokens.astype(jnp.int32), positions.astype(jnp.int32),
                types.astype(jnp.int32), wte, wpe, tte, ln_w, ln_b)
